# baseline (speedup 1.0000x reference)
import functools

import jax
import jax.numpy as jnp
from jax import lax
from jax.experimental import pallas as pl
from jax.experimental.pallas import tpu as pltpu
from jax.experimental.pallas import tpu_sc as plsc

_N = 10000
_E = 320000
_DN = 128
_DE = 16
_H = 128

_CH = 128
_NCHUNK = _E // _CH
_PW = 80
_HF = 64
_NC = 2
_NS = 16
_NPAD = 10240
_RPT = _NPAD // _NS
_ZR = 128


def _preh_body(x_ref, w_ref, h_ref):
    h_ref[...] = jnp.dot(x_ref[...], w_ref[...],
                         preferred_element_type=jnp.float32)


def _preh_call(x, w):
    blk = 1000
    return pl.pallas_call(
        _preh_body,
        grid=(_N // blk,),
        in_specs=[pl.BlockSpec((blk, _DN), lambda i: (i, 0)),
                  pl.BlockSpec((_DN, _H), lambda i: (0, 0))],
        out_specs=pl.BlockSpec((blk, _H), lambda i: (i, 0)),
        out_shape=jax.ShapeDtypeStruct((_N, _H), jnp.float32),
    )(x, w)


def _pree_body(e_ref, w_ref, b_ref, o_ref):
    o_ref[...] = jnp.dot(e_ref[...], w_ref[...],
                         preferred_element_type=jnp.float32) + b_ref[...]


def _pree_call(e, w, b):
    blk = 8000
    return pl.pallas_call(
        _pree_body,
        grid=(_E // blk,),
        in_specs=[pl.BlockSpec((blk, _DE), lambda i: (i, 0)),
                  pl.BlockSpec((_DE, _H), lambda i: (0, 0)),
                  pl.BlockSpec((1, _H), lambda i: (0, 0))],
        out_specs=pl.BlockSpec((blk, _H), lambda i: (i, 0)),
        out_shape=jax.ShapeDtypeStruct((_E, _H), jnp.float32),
    )(e, w, b)


def _epi_body(p_ref, x_ref, w1b_ref, b1b_ref, w2a_ref, b2a_ref, w2b_ref,
              b2b_ref, o_ref):
    sa = p_ref[0]
    sb = p_ref[1]
    cnt = sa[:, _HF:_HF + 1]
    ssum = jnp.concatenate([sa[:, :_HF], sb[:, :_HF]], axis=1)
    mean = ssum / jnp.maximum(cnt, 1.0)
    agg = (jnp.dot(mean, w1b_ref[...], preferred_element_type=jnp.float32)
           + b1b_ref[...] * (cnt > 0.0).astype(jnp.float32))
    xb = x_ref[...]
    h = jnp.maximum(
        jnp.dot(xb, w2a_ref[:_DN], preferred_element_type=jnp.float32)
        + jnp.dot(agg, w2a_ref[_DN:], preferred_element_type=jnp.float32)
        + b2a_ref[...], 0.0)
    o_ref[...] = (jnp.dot(h, w2b_ref[...], preferred_element_type=jnp.float32)
                  + b2b_ref[...] + xb)


def _epi_call(part, x, w1b, b1b, w2a, b2a, w2b, b2b):
    blk = 1000
    return pl.pallas_call(
        _epi_body,
        grid=(_N // blk,),
        in_specs=[pl.BlockSpec((2, blk, _PW), lambda i: (0, i, 0)),
                  pl.BlockSpec((blk, _DN), lambda i: (i, 0)),
                  pl.BlockSpec((_H, _H), lambda i: (0, 0)),
                  pl.BlockSpec((1, _H), lambda i: (0, 0)),
                  pl.BlockSpec((_H + _DN, _H), lambda i: (0, 0)),
                  pl.BlockSpec((1, _H), lambda i: (0, 0)),
                  pl.BlockSpec((_H, _DN), lambda i: (0, 0)),
                  pl.BlockSpec((1, _DN), lambda i: (0, 0))],
        out_specs=pl.BlockSpec((blk, _DN), lambda i: (i, 0)),
        out_shape=jax.ShapeDtypeStruct((_N, _DN), jnp.float32),
    )(part, x, w1b, b1b, w2a, b2a, w2b, b2b)


_MESH = plsc.VectorSubcoreMesh(core_axis_name="c", subcore_axis_name="s")


@functools.partial(
    pl.kernel,
    out_type=jax.ShapeDtypeStruct((_NC, _NPAD, _PW), jnp.float32),
    mesh=_MESH,
    scratch_types=[
        pltpu.VMEM((_CH,), jnp.int32),
        pltpu.VMEM((_CH,), jnp.int32),
        pltpu.VMEM((_CH, _H), jnp.float32),
        pltpu.VMEM((_CH, _H), jnp.float32),
        pltpu.VMEM((_CH, _PW), jnp.float32),
        pltpu.VMEM((_ZR, _PW), jnp.float32),
        pltpu.VMEM_SHARED((_NPAD, _PW), jnp.float32),
        pltpu.SemaphoreType.DMA,
    ],
    compiler_params=pltpu.CompilerParams(use_tc_tiling_on_sc=False))
def _sc_edge(h1_hbm, ew_hbm, row_hbm, col_hbm, out_hbm,
             colv, rowv, ewbuf, hbuf, ebuf, zbuf, seg, sem):
    cid = lax.axis_index("c")
    sid = lax.axis_index("s")
    cbase = cid * _HF

    def zrow(r, carry):
        for g in range(_PW // 16):
            zbuf[r, pl.ds(g * 16, 16)] = jnp.zeros((16,), jnp.float32)
        return carry

    lax.fori_loop(0, _ZR, zrow, 0)
    for k in range(_RPT // _ZR):
        pltpu.sync_copy(zbuf, seg.at[pl.ds(sid * _RPT + k * _ZR, _ZR)])

    cpad = jnp.where(lax.iota(jnp.int32, 16) == 0, 1.0, 0.0)

    def crow(r, carry):
        ebuf[r, pl.ds(_HF, 16)] = cpad
        return carry

    lax.fori_loop(0, _CH, crow, 0)
    plsc.subcore_barrier()

    tmax = (_NCHUNK + _NS - 1) // _NS

    def chunk_body(t, carry):
        chunk = sid + t * _NS

        @pl.when(chunk < _NCHUNK)
        def _():
            off = chunk * _CH
            pltpu.sync_copy(col_hbm.at[pl.ds(off, _CH)], colv)
            pltpu.sync_copy(row_hbm.at[pl.ds(off, _CH)], rowv)
            pltpu.sync_copy(ew_hbm.at[pl.ds(off, _CH)], ewbuf)
            pltpu.async_copy(h1_hbm.at[colv], hbuf, sem).wait()

            def rbody(r, c2):
                for g in range(_HF // 16):
                    src = pl.ds(cbase + g * 16, 16)
                    ebuf[r, pl.ds(g * 16, 16)] = jnp.maximum(
                        ewbuf[r, src] + hbuf[r, src], 0.0)
                return c2

            lax.fori_loop(0, _CH, rbody, 0)
            pltpu.sync_copy(ebuf, seg.at[rowv], add=True)

        return carry

    lax.fori_loop(0, tmax, chunk_body, 0)
    plsc.subcore_barrier()
    pltpu.sync_copy(seg.at[pl.ds(sid * _RPT, _RPT)],
                    out_hbm.at[cid, pl.ds(sid * _RPT, _RPT)])


def kernel(x, edge_index, edge_attr, u, batch,
           W1a, b1a, W1b, b1b, W2a, b2a, W2b, b2b):
    del u, batch
    row = edge_index[0]
    col = edge_index[1]
    h1 = _preh_call(x, W1a[:_DN])
    ew = _pree_call(edge_attr, W1a[_DN:], b1a.reshape(1, _H))
    part = _sc_edge(h1, ew, row, col)
    return _epi_call(part, x, W1b, b1b.reshape(1, _H), W2a,
                     b2a.reshape(1, _H), W2b, b2b.reshape(1, _DN))


# speedup vs baseline: 1.2877x; 1.2877x over previous
import functools

import jax
import jax.numpy as jnp
from jax import lax
from jax.experimental import pallas as pl
from jax.experimental.pallas import tpu as pltpu
from jax.experimental.pallas import tpu_sc as plsc

_N = 10000
_E = 320000
_DN = 128
_DE = 16
_H = 128

_CH = 128
_PW = 80
_HF = 64
_NC = 2
_NS = 16
_NPAD = 10240
_RPT = _NPAD // _NS
_ZR = 128
_TMAX = 157
_EPC = _TMAX * _NS
_EP = _EPC * _CH
_DUMP = _NPAD - 1


def _preh_body(x_ref, w_ref, h_ref):
    h_ref[...] = jnp.dot(x_ref[...], w_ref[...],
                         preferred_element_type=jnp.float32)


def _preh_call(x, w):
    blk = 1000
    return pl.pallas_call(
        _preh_body,
        grid=(_N // blk,),
        in_specs=[pl.BlockSpec((blk, _DN), lambda i: (i, 0)),
                  pl.BlockSpec((_DN, _H), lambda i: (0, 0))],
        out_specs=pl.BlockSpec((blk, _H), lambda i: (i, 0)),
        out_shape=jax.ShapeDtypeStruct((_N, _H), jnp.float32),
    )(x, w)


def _pree_body(e_ref, w_ref, b_ref, o_ref):
    o_ref[...] = jnp.dot(e_ref[...], w_ref[...],
                         preferred_element_type=jnp.float32) + b_ref[...]


def _pree_call(e, w, b):
    blk = 2048
    return pl.pallas_call(
        _pree_body,
        grid=(_EP // blk,),
        in_specs=[pl.BlockSpec((blk, _DE), lambda i: (i, 0)),
                  pl.BlockSpec((_DE, _H), lambda i: (0, 0)),
                  pl.BlockSpec((1, _H), lambda i: (0, 0))],
        out_specs=pl.BlockSpec((blk, _H), lambda i: (i, 0)),
        out_shape=jax.ShapeDtypeStruct((_EP, _H), jnp.float32),
    )(e, w, b)


def _epi_body(p_ref, x_ref, w1b_ref, b1b_ref, w2a_ref, b2a_ref, w2b_ref,
              b2b_ref, o_ref):
    sa = p_ref[0]
    sb = p_ref[1]
    cnt = sa[:, _HF:_HF + 1]
    ssum = jnp.concatenate([sa[:, :_HF], sb[:, :_HF]], axis=1)
    mean = ssum / jnp.maximum(cnt, 1.0)
    agg = (jnp.dot(mean, w1b_ref[...], preferred_element_type=jnp.float32)
           + b1b_ref[...] * (cnt > 0.0).astype(jnp.float32))
    xb = x_ref[...]
    h = jnp.maximum(
        jnp.dot(xb, w2a_ref[:_DN], preferred_element_type=jnp.float32)
        + jnp.dot(agg, w2a_ref[_DN:], preferred_element_type=jnp.float32)
        + b2a_ref[...], 0.0)
    o_ref[...] = (jnp.dot(h, w2b_ref[...], preferred_element_type=jnp.float32)
                  + b2b_ref[...] + xb)


def _epi_call(part, x, w1b, b1b, w2a, b2a, w2b, b2b):
    blk = 1000
    return pl.pallas_call(
        _epi_body,
        grid=(_N // blk,),
        in_specs=[pl.BlockSpec((2, blk, _PW), lambda i: (0, i, 0)),
                  pl.BlockSpec((blk, _DN), lambda i: (i, 0)),
                  pl.BlockSpec((_H, _H), lambda i: (0, 0)),
                  pl.BlockSpec((1, _H), lambda i: (0, 0)),
                  pl.BlockSpec((_H + _DN, _H), lambda i: (0, 0)),
                  pl.BlockSpec((1, _H), lambda i: (0, 0)),
                  pl.BlockSpec((_H, _DN), lambda i: (0, 0)),
                  pl.BlockSpec((1, _DN), lambda i: (0, 0))],
        out_specs=pl.BlockSpec((blk, _DN), lambda i: (i, 0)),
        out_shape=jax.ShapeDtypeStruct((_N, _DN), jnp.float32),
    )(part, x, w1b, b1b, w2a, b2a, w2b, b2b)


_MESH = plsc.VectorSubcoreMesh(core_axis_name="c", subcore_axis_name="s")


@functools.partial(
    pl.kernel,
    out_type=jax.ShapeDtypeStruct((_NC, _NPAD, _PW), jnp.float32),
    mesh=_MESH,
    scratch_types=[
        pltpu.VMEM((_CH,), jnp.int32),
        pltpu.VMEM((_CH,), jnp.int32),
        pltpu.VMEM((_CH,), jnp.int32),
        pltpu.VMEM((_CH,), jnp.int32),
        pltpu.VMEM((_CH,), jnp.int32),
        pltpu.VMEM((_CH,), jnp.int32),
        pltpu.VMEM((_CH, _HF), jnp.float32),
        pltpu.VMEM((_CH, _HF), jnp.float32),
        pltpu.VMEM((_CH, _H), jnp.float32),
        pltpu.VMEM((_CH, _H), jnp.float32),
        pltpu.VMEM((_CH, _PW), jnp.float32),
        pltpu.VMEM((_CH, _PW), jnp.float32),
        pltpu.VMEM_SHARED((_NPAD, _PW), jnp.float32),
        pltpu.SemaphoreType.DMA,
        pltpu.SemaphoreType.DMA,
        pltpu.SemaphoreType.DMA,
        pltpu.SemaphoreType.DMA,
        pltpu.SemaphoreType.DMA,
        pltpu.SemaphoreType.DMA,
    ],
    compiler_params=pltpu.CompilerParams(use_tc_tiling_on_sc=False))
def _sc_edge(h1_hbm, ew_hbm, row_hbm, col_hbm, out_hbm,
             colv0, colv1, rowa0, rowa1, rows0, rows1, ewb0, ewb1,
             hb0, hb1, eb0, eb1, seg,
             semA0, semA1, semG0, semG1, semS0, semS1):
    cid = lax.axis_index("c")
    sid = lax.axis_index("s")
    cbase = cid * _HF

    slots = ((colv0, rowa0, rows0, ewb0, hb0, eb0, semA0, semG0, semS0),
             (colv1, rowa1, rows1, ewb1, hb1, eb1, semA1, semG1, semS1))

    def zrow(r, carry):
        for g in range(_PW // 16):
            eb0[r, pl.ds(g * 16, 16)] = jnp.zeros((16,), jnp.float32)
        return carry

    lax.fori_loop(0, _CH, zrow, 0)
    for k in range(_RPT // _ZR):
        pltpu.sync_copy(eb0, seg.at[pl.ds(sid * _RPT + k * _ZR, _ZR)])

    cpad = jnp.where(lax.iota(jnp.int32, 16) == 0, 1.0, 0.0)

    def crow(r, carry):
        eb0[r, pl.ds(_HF, 16)] = cpad
        eb1[r, pl.ds(_HF, 16)] = cpad
        return carry

    lax.fori_loop(0, _CH, crow, 0)
    plsc.subcore_barrier()

    def a_issue(t, s):
        colv, rowa, rows_, ewb, hb, eb, semA, semG, semS = s
        off = (sid + t * _NS) * _CH
        pltpu.async_copy(col_hbm.at[pl.ds(off, _CH)], colv, semA)
        pltpu.async_copy(row_hbm.at[pl.ds(off, _CH)], rowa, semA)
        pltpu.async_copy(ew_hbm.at[pl.ds(off, _CH), pl.ds(cbase, _HF)],
                         ewb, semA)

    def a_wait(s):
        colv, rowa, rows_, ewb, hb, eb, semA, semG, semS = s
        pltpu.make_async_copy(col_hbm.at[pl.ds(0, _CH)], colv, semA).wait()
        pltpu.make_async_copy(row_hbm.at[pl.ds(0, _CH)], rowa, semA).wait()
        pltpu.make_async_copy(ew_hbm.at[pl.ds(0, _CH), pl.ds(cbase, _HF)],
                              ewb, semA).wait()

    def g_issue(s):
        colv, rowa, rows_, ewb, hb, eb, semA, semG, semS = s
        pltpu.async_copy(h1_hbm.at[colv], hb, semG)

    def g_wait(s):
        colv, rowa, rows_, ewb, hb, eb, semA, semG, semS = s
        pltpu.make_async_copy(h1_hbm.at[colv], hb, semG).wait()

    def s_issue(s):
        colv, rowa, rows_, ewb, hb, eb, semA, semG, semS = s
        pltpu.async_copy(eb, seg.at[rows_], semS, add=True)

    def s_wait(s):
        colv, rowa, rows_, ewb, hb, eb, semA, semG, semS = s
        pltpu.make_async_copy(eb, seg.at[rows_], semS).wait()

    def compute(s):
        colv, rowa, rows_, ewb, hb, eb, semA, semG, semS = s
        for g in range(_CH // 16):
            rows_[pl.ds(g * 16, 16)] = rowa[pl.ds(g * 16, 16)]

        def rbody(r, c2):
            for g in range(_HF // 16):
                sl = pl.ds(g * 16, 16)
                eb[r, sl] = jnp.maximum(
                    ewb[r, sl] + hb[r, pl.ds(cbase + g * 16, 16)], 0.0)
            return c2

        lax.fori_loop(0, _CH, rbody, 0)

    def step(t, b):
        s, ns = slots[b], slots[1 - b]
        g_wait(s)
        a_wait(ns)
        g_issue(ns)

        @pl.when(t >= 2)
        def _():
            s_wait(s)

        compute(s)
        s_issue(s)

        @pl.when(t <= _TMAX - 3)
        def _():
            a_issue(t + 2, s)

    a_issue(0, slots[0])
    a_issue(1, slots[1])
    a_wait(slots[0])
    g_issue(slots[0])

    def pair_body(u, carry):
        t = u * 2
        step(t, 0)
        step(t + 1, 1)
        return carry

    lax.fori_loop(0, (_TMAX - 1) // 2, pair_body, 0)

    s0 = slots[0]
    g_wait(s0)
    s_wait(s0)
    compute(s0)
    s_issue(s0)
    s_wait(slots[1])
    s_wait(slots[0])

    plsc.subcore_barrier()
    pltpu.sync_copy(seg.at[pl.ds(sid * _RPT, _RPT)],
                    out_hbm.at[cid, pl.ds(sid * _RPT, _RPT)])


def kernel(x, edge_index, edge_attr, u, batch,
           W1a, b1a, W1b, b1b, W2a, b2a, W2b, b2b):
    del u, batch
    npad = _EP - _E
    row = jnp.concatenate([edge_index[0],
                           jnp.full((npad,), _DUMP, jnp.int32)])
    col = jnp.concatenate([edge_index[1], jnp.zeros((npad,), jnp.int32)])
    ea = jnp.concatenate([edge_attr, jnp.zeros((npad, _DE), jnp.float32)])
    h1 = _preh_call(x, W1a[:_DN])
    ew = _pree_call(ea, W1a[_DN:], b1a.reshape(1, _H))
    part = _sc_edge(h1, ew, row, col)
    return _epi_call(part, x, W1b, b1b.reshape(1, _H), W2a,
                     b2a.reshape(1, _H), W2b, b2b.reshape(1, _DN))


# speedup vs baseline: 1.8840x; 1.4631x over previous
import functools

import jax
import jax.numpy as jnp
from jax import lax
from jax.experimental import pallas as pl
from jax.experimental.pallas import tpu as pltpu
from jax.experimental.pallas import tpu_sc as plsc

_N = 10000
_E = 320000
_DN = 128
_DE = 16
_H = 128

_CH = 128
_PW = 80
_HF = 64
_NC = 2
_NS = 16
_NPAD = 10240
_RPT = _NPAD // _NS
_ZR = 128
_TMAX = 157
_EPC = _TMAX * _NS
_EP = _EPC * _CH
_DUMP = _NPAD - 1


def _preh_body(x_ref, w_ref, h_ref):
    h_ref[...] = jnp.dot(x_ref[...], w_ref[...],
                         preferred_element_type=jnp.float32)


def _preh_call(x, w):
    blk = 1000
    return pl.pallas_call(
        _preh_body,
        grid=(_N // blk,),
        in_specs=[pl.BlockSpec((blk, _DN), lambda i: (i, 0)),
                  pl.BlockSpec((_DN, _H), lambda i: (0, 0))],
        out_specs=pl.BlockSpec((blk, _H), lambda i: (i, 0)),
        out_shape=jax.ShapeDtypeStruct((_N, _H), jnp.float32),
    )(x, w)


def _pree_body(e_ref, w_ref, b_ref, o_ref):
    o_ref[...] = jnp.dot(e_ref[...], w_ref[...],
                         preferred_element_type=jnp.float32) + b_ref[...]


def _pree_call(e, w, b):
    blk = 2048
    return pl.pallas_call(
        _pree_body,
        grid=(_EP // blk,),
        in_specs=[pl.BlockSpec((blk, _DE), lambda i: (i, 0)),
                  pl.BlockSpec((_DE, _H), lambda i: (0, 0)),
                  pl.BlockSpec((1, _H), lambda i: (0, 0))],
        out_specs=pl.BlockSpec((blk, _H), lambda i: (i, 0)),
        out_shape=jax.ShapeDtypeStruct((_EP, _H), jnp.float32),
    )(e, w, b)


def _epi_body(p_ref, x_ref, w1b_ref, b1b_ref, w2a_ref, b2a_ref, w2b_ref,
              b2b_ref, o_ref):
    sa = p_ref[0]
    sb = p_ref[1]
    cnt = sa[:, _HF:_HF + 1]
    ssum = jnp.concatenate([sa[:, :_HF], sb[:, :_HF]], axis=1)
    mean = ssum / jnp.maximum(cnt, 1.0)
    agg = (jnp.dot(mean, w1b_ref[...], preferred_element_type=jnp.float32)
           + b1b_ref[...] * (cnt > 0.0).astype(jnp.float32))
    xb = x_ref[...]
    h = jnp.maximum(
        jnp.dot(xb, w2a_ref[:_DN], preferred_element_type=jnp.float32)
        + jnp.dot(agg, w2a_ref[_DN:], preferred_element_type=jnp.float32)
        + b2a_ref[...], 0.0)
    o_ref[...] = (jnp.dot(h, w2b_ref[...], preferred_element_type=jnp.float32)
                  + b2b_ref[...] + xb)


def _epi_call(part, x, w1b, b1b, w2a, b2a, w2b, b2b):
    blk = 1000
    return pl.pallas_call(
        _epi_body,
        grid=(_N // blk,),
        in_specs=[pl.BlockSpec((2, blk, _PW), lambda i: (0, i, 0)),
                  pl.BlockSpec((blk, _DN), lambda i: (i, 0)),
                  pl.BlockSpec((_H, _H), lambda i: (0, 0)),
                  pl.BlockSpec((1, _H), lambda i: (0, 0)),
                  pl.BlockSpec((_H + _DN, _H), lambda i: (0, 0)),
                  pl.BlockSpec((1, _H), lambda i: (0, 0)),
                  pl.BlockSpec((_H, _DN), lambda i: (0, 0)),
                  pl.BlockSpec((1, _DN), lambda i: (0, 0))],
        out_specs=pl.BlockSpec((blk, _DN), lambda i: (i, 0)),
        out_shape=jax.ShapeDtypeStruct((_N, _DN), jnp.float32),
    )(part, x, w1b, b1b, w2a, b2a, w2b, b2b)


_MESH = plsc.VectorSubcoreMesh(core_axis_name="c", subcore_axis_name="s")


@functools.partial(
    pl.kernel,
    out_type=jax.ShapeDtypeStruct((_NC, _NPAD, _PW), jnp.float32),
    mesh=_MESH,
    scratch_types=[
        pltpu.VMEM((_CH,), jnp.int32),
        pltpu.VMEM((_CH,), jnp.int32),
        pltpu.VMEM((_CH,), jnp.int32),
        pltpu.VMEM((_CH,), jnp.int32),
        pltpu.VMEM((_CH,), jnp.int32),
        pltpu.VMEM((_CH,), jnp.int32),
        pltpu.VMEM((_CH, _HF), jnp.float32),
        pltpu.VMEM((_CH, _HF), jnp.float32),
        pltpu.VMEM((_CH, _H), jnp.float32),
        pltpu.VMEM((_CH, _H), jnp.float32),
        pltpu.VMEM((_CH, _PW), jnp.float32),
        pltpu.VMEM((_CH, _PW), jnp.float32),
        pltpu.VMEM_SHARED((_NPAD, _PW), jnp.float32),
        pltpu.SemaphoreType.DMA,
        pltpu.SemaphoreType.DMA,
        pltpu.SemaphoreType.DMA,
        pltpu.SemaphoreType.DMA,
        pltpu.SemaphoreType.DMA,
        pltpu.SemaphoreType.DMA,
    ],
    compiler_params=pltpu.CompilerParams(use_tc_tiling_on_sc=False))
def _sc_edge(h1_hbm, ew_hbm, row_hbm, col_hbm, out_hbm,
             colv0, colv1, rowa0, rowa1, rows0, rows1, ewb0, ewb1,
             hb0, hb1, eb0, eb1, seg,
             semA0, semA1, semG0, semG1, semS0, semS1):
    cid = lax.axis_index("c")
    sid = lax.axis_index("s")
    cbase = cid * _HF

    slots = ((colv0, rowa0, rows0, ewb0, hb0, eb0, semA0, semG0, semS0),
             (colv1, rowa1, rows1, ewb1, hb1, eb1, semA1, semG1, semS1))

    def zrow(r, carry):
        for g in range(_PW // 16):
            eb0[r, pl.ds(g * 16, 16)] = jnp.zeros((16,), jnp.float32)
        return carry

    lax.fori_loop(0, _CH, zrow, 0)
    for k in range(_RPT // _ZR):
        pltpu.sync_copy(eb0, seg.at[pl.ds(sid * _RPT + k * _ZR, _ZR)])

    cpad = jnp.where(lax.iota(jnp.int32, 16) == 0, 1.0, 0.0)

    def crow(r, carry):
        eb0[r, pl.ds(_HF, 16)] = cpad
        eb1[r, pl.ds(_HF, 16)] = cpad
        return carry

    lax.fori_loop(0, _CH, crow, 0)
    plsc.subcore_barrier()

    def a_issue(t, s):
        colv, rowa, rows_, ewb, hb, eb, semA, semG, semS = s
        off = (sid + t * _NS) * _CH
        pltpu.async_copy(col_hbm.at[pl.ds(off, _CH)], colv, semA)
        pltpu.async_copy(row_hbm.at[pl.ds(off, _CH)], rowa, semA)
        pltpu.async_copy(ew_hbm.at[pl.ds(off, _CH), pl.ds(cbase, _HF)],
                         ewb, semA)

    def a_wait(s):
        colv, rowa, rows_, ewb, hb, eb, semA, semG, semS = s
        pltpu.make_async_copy(col_hbm.at[pl.ds(0, _CH)], colv, semA).wait()
        pltpu.make_async_copy(row_hbm.at[pl.ds(0, _CH)], rowa, semA).wait()
        pltpu.make_async_copy(ew_hbm.at[pl.ds(0, _CH), pl.ds(cbase, _HF)],
                              ewb, semA).wait()

    def g_issue(s):
        colv, rowa, rows_, ewb, hb, eb, semA, semG, semS = s
        pltpu.async_copy(h1_hbm.at[colv], hb, semG)

    def g_wait(s):
        colv, rowa, rows_, ewb, hb, eb, semA, semG, semS = s
        pltpu.make_async_copy(h1_hbm.at[colv], hb, semG).wait()

    def s_issue(s):
        colv, rowa, rows_, ewb, hb, eb, semA, semG, semS = s
        pltpu.async_copy(eb, seg.at[rows_], semS, add=True)

    def s_wait(s):
        colv, rowa, rows_, ewb, hb, eb, semA, semG, semS = s
        pltpu.make_async_copy(eb, seg.at[rows_], semS).wait()

    def compute(s):
        colv, rowa, rows_, ewb, hb, eb, semA, semG, semS = s
        for g in range(_CH // 16):
            rows_[pl.ds(g * 16, 16)] = rowa[pl.ds(g * 16, 16)]

        @plsc.parallel_loop(0, _CH, unroll=4)
        def rbody(r):
            for g in range(_HF // 16):
                sl = pl.ds(g * 16, 16)
                eb[r, sl] = jnp.maximum(
                    ewb[r, sl] + hb[r, pl.ds(cbase + g * 16, 16)], 0.0)

    def step(t, b):
        s, ns = slots[b], slots[1 - b]
        g_wait(s)
        a_wait(ns)
        g_issue(ns)

        @pl.when(t >= 2)
        def _():
            s_wait(s)

        compute(s)
        s_issue(s)

        @pl.when(t <= _TMAX - 3)
        def _():
            a_issue(t + 2, s)

    a_issue(0, slots[0])
    a_issue(1, slots[1])
    a_wait(slots[0])
    g_issue(slots[0])

    def pair_body(u, carry):
        t = u * 2
        step(t, 0)
        step(t + 1, 1)
        return carry

    lax.fori_loop(0, (_TMAX - 1) // 2, pair_body, 0)

    s0 = slots[0]
    g_wait(s0)
    s_wait(s0)
    compute(s0)
    s_issue(s0)
    s_wait(slots[1])
    s_wait(slots[0])

    plsc.subcore_barrier()
    pltpu.sync_copy(seg.at[pl.ds(sid * _RPT, _RPT)],
                    out_hbm.at[cid, pl.ds(sid * _RPT, _RPT)])


def kernel(x, edge_index, edge_attr, u, batch,
           W1a, b1a, W1b, b1b, W2a, b2a, W2b, b2b):
    del u, batch
    npad = _EP - _E
    row = jnp.concatenate([edge_index[0],
                           jnp.full((npad,), _DUMP, jnp.int32)])
    col = jnp.concatenate([edge_index[1], jnp.zeros((npad,), jnp.int32)])
    ea = jnp.concatenate([edge_attr, jnp.zeros((npad, _DE), jnp.float32)])
    h1 = _preh_call(x, W1a[:_DN])
    ew = _pree_call(ea, W1a[_DN:], b1a.reshape(1, _H))
    part = _sc_edge(h1, ew, row, col)
    return _epi_call(part, x, W1b, b1b.reshape(1, _H), W2a,
                     b2a.reshape(1, _H), W2b, b2b.reshape(1, _DN))


# speedup vs baseline: 2.6224x; 1.3919x over previous
import functools

import jax
import jax.numpy as jnp
from jax import lax
from jax.experimental import pallas as pl
from jax.experimental.pallas import tpu as pltpu
from jax.experimental.pallas import tpu_sc as plsc

_N = 10000
_E = 320000
_DN = 128
_DE = 16
_H = 128

_CH = 128
_PW = 72
_HF = 64
_NC = 2
_NS = 16
_NPAD = 10240
_RPT = _NPAD // _NS
_ZR = 128
_NCH = _E // _CH
_TMAX = 157
_DUMP = _NPAD - 1


def _preh_body(x_ref, w_ref, h_ref):
    h_ref[...] = jnp.dot(x_ref[...], w_ref[...],
                         preferred_element_type=jnp.float32)


def _preh_call(x, w):
    blk = 1000
    return pl.pallas_call(
        _preh_body,
        grid=(_N // blk,),
        in_specs=[pl.BlockSpec((blk, _DN), lambda i: (i, 0)),
                  pl.BlockSpec((_DN, _H), lambda i: (0, 0))],
        out_specs=pl.BlockSpec((blk, _H), lambda i: (i, 0)),
        out_shape=jax.ShapeDtypeStruct((_N, _H), jnp.float32),
    )(x, w)


def _pree_body(e_ref, w_ref, b_ref, o_ref):
    o_ref[...] = jnp.dot(e_ref[...], w_ref[...],
                         preferred_element_type=jnp.float32) + b_ref[...]


def _pree_call(e, w, b):
    blk = 8000
    return pl.pallas_call(
        _pree_body,
        grid=(_E // blk,),
        in_specs=[pl.BlockSpec((blk, _DE), lambda i: (i, 0)),
                  pl.BlockSpec((_DE, _H), lambda i: (0, 0)),
                  pl.BlockSpec((1, _H), lambda i: (0, 0))],
        out_specs=pl.BlockSpec((blk, _H), lambda i: (i, 0)),
        out_shape=jax.ShapeDtypeStruct((_E, _H), jnp.float32),
    )(e, w, b)


def _epi_body(p_ref, x_ref, w1b_ref, b1b_ref, w2a_ref, b2a_ref, w2b_ref,
              b2b_ref, o_ref):
    sa = p_ref[0]
    sb = p_ref[1]
    cnt = sa[:, _HF:_HF + 1]
    ssum = jnp.concatenate([sa[:, :_HF], sb[:, :_HF]], axis=1)
    mean = ssum / jnp.maximum(cnt, 1.0)
    agg = (jnp.dot(mean, w1b_ref[...], preferred_element_type=jnp.float32)
           + b1b_ref[...] * (cnt > 0.0).astype(jnp.float32))
    xb = x_ref[...]
    h = jnp.maximum(
        jnp.dot(xb, w2a_ref[:_DN], preferred_element_type=jnp.float32)
        + jnp.dot(agg, w2a_ref[_DN:], preferred_element_type=jnp.float32)
        + b2a_ref[...], 0.0)
    o_ref[...] = (jnp.dot(h, w2b_ref[...], preferred_element_type=jnp.float32)
                  + b2b_ref[...] + xb)


def _epi_call(part, x, w1b, b1b, w2a, b2a, w2b, b2b):
    blk = 1000
    return pl.pallas_call(
        _epi_body,
        grid=(_N // blk,),
        in_specs=[pl.BlockSpec((2, blk, _PW), lambda i: (0, i, 0)),
                  pl.BlockSpec((blk, _DN), lambda i: (i, 0)),
                  pl.BlockSpec((_H, _H), lambda i: (0, 0)),
                  pl.BlockSpec((1, _H), lambda i: (0, 0)),
                  pl.BlockSpec((_H + _DN, _H), lambda i: (0, 0)),
                  pl.BlockSpec((1, _H), lambda i: (0, 0)),
                  pl.BlockSpec((_H, _DN), lambda i: (0, 0)),
                  pl.BlockSpec((1, _DN), lambda i: (0, 0))],
        out_specs=pl.BlockSpec((blk, _DN), lambda i: (i, 0)),
        out_shape=jax.ShapeDtypeStruct((_N, _DN), jnp.float32),
    )(part, x, w1b, b1b, w2a, b2a, w2b, b2b)


_MESH = plsc.VectorSubcoreMesh(core_axis_name="c", subcore_axis_name="s")


@functools.partial(
    pl.kernel,
    out_type=jax.ShapeDtypeStruct((_NC, _NPAD, _PW), jnp.float32),
    mesh=_MESH,
    scratch_types=[
        pltpu.VMEM((_CH,), jnp.int32),
        pltpu.VMEM((_CH,), jnp.int32),
        pltpu.VMEM((_CH,), jnp.int32),
        pltpu.VMEM((_CH,), jnp.int32),
        pltpu.VMEM((_CH,), jnp.int32),
        pltpu.VMEM((_CH,), jnp.int32),
        pltpu.VMEM((_CH, _HF), jnp.float32),
        pltpu.VMEM((_CH, _HF), jnp.float32),
        pltpu.VMEM((_CH, _H), jnp.float32),
        pltpu.VMEM((_CH, _H), jnp.float32),
        pltpu.VMEM((_CH, _PW), jnp.float32),
        pltpu.VMEM((_CH, _PW), jnp.float32),
        pltpu.VMEM_SHARED((_NPAD, _PW), jnp.float32),
        pltpu.SemaphoreType.DMA,
        pltpu.SemaphoreType.DMA,
        pltpu.SemaphoreType.DMA,
        pltpu.SemaphoreType.DMA,
        pltpu.SemaphoreType.DMA,
        pltpu.SemaphoreType.DMA,
    ],
    compiler_params=pltpu.CompilerParams(use_tc_tiling_on_sc=False))
def _sc_edge(h1_hbm, ew_hbm, ei_hbm, out_hbm,
             colv0, colv1, rowa0, rowa1, rows0, rows1, ewb0, ewb1,
             hb0, hb1, eb0, eb1, seg,
             semA0, semA1, semG0, semG1, semS0, semS1):
    cid = lax.axis_index("c")
    sid = lax.axis_index("s")
    cbase = cid * _HF

    slots = ((colv0, rowa0, rows0, ewb0, hb0, eb0, semA0, semG0, semS0),
             (colv1, rowa1, rows1, ewb1, hb1, eb1, semA1, semG1, semS1))

    def zrow(r, carry):
        for g in range(_PW // 16):
            eb0[r, pl.ds(g * 16, 16)] = jnp.zeros((16,), jnp.float32)
        return carry

    lax.fori_loop(0, _CH, zrow, 0)
    for k in range(_RPT // _ZR):
        pltpu.sync_copy(eb0, seg.at[pl.ds(sid * _RPT + k * _ZR, _ZR)])

    cpad = jnp.where(lax.iota(jnp.int32, 16) == 8, 1.0, 0.0)

    def crow(r, carry):
        eb0[r, pl.ds(_PW - 16, 16)] = cpad
        eb1[r, pl.ds(_PW - 16, 16)] = cpad
        return carry

    lax.fori_loop(0, _CH, crow, 0)
    plsc.subcore_barrier()

    def a_issue(t, s):
        colv, rowa, rows_, ewb, hb, eb, semA, semG, semS = s
        off = jnp.minimum((sid + t * _NS) * _CH, _E - _CH)
        pltpu.async_copy(ei_hbm.at[1, pl.ds(off, _CH)], colv, semA)
        pltpu.async_copy(ei_hbm.at[0, pl.ds(off, _CH)], rowa, semA)
        pltpu.async_copy(ew_hbm.at[pl.ds(off, _CH), pl.ds(cbase, _HF)],
                         ewb, semA)

    def a_wait(s):
        colv, rowa, rows_, ewb, hb, eb, semA, semG, semS = s
        pltpu.make_async_copy(ei_hbm.at[1, pl.ds(0, _CH)], colv, semA).wait()
        pltpu.make_async_copy(ei_hbm.at[0, pl.ds(0, _CH)], rowa, semA).wait()
        pltpu.make_async_copy(ew_hbm.at[pl.ds(0, _CH), pl.ds(cbase, _HF)],
                              ewb, semA).wait()

    def g_issue(s):
        colv, rowa, rows_, ewb, hb, eb, semA, semG, semS = s
        pltpu.async_copy(h1_hbm.at[colv], hb, semG)

    def g_wait(s):
        colv, rowa, rows_, ewb, hb, eb, semA, semG, semS = s
        pltpu.make_async_copy(h1_hbm.at[colv], hb, semG).wait()

    def s_issue(s):
        colv, rowa, rows_, ewb, hb, eb, semA, semG, semS = s
        pltpu.async_copy(eb, seg.at[rows_], semS, add=True)

    def s_wait(s):
        colv, rowa, rows_, ewb, hb, eb, semA, semG, semS = s
        pltpu.make_async_copy(eb, seg.at[rows_], semS).wait()

    def compute(t, s):
        colv, rowa, rows_, ewb, hb, eb, semA, semG, semS = s
        is_pad = (sid + t * _NS) >= _NCH
        dump = jnp.full((16,), _DUMP, jnp.int32)
        for g in range(_CH // 16):
            sl = pl.ds(g * 16, 16)
            rows_[sl] = jnp.where(is_pad, dump, rowa[sl])

        @plsc.parallel_loop(0, _CH, unroll=4)
        def rbody(r):
            for g in range(_HF // 16):
                sl = pl.ds(g * 16, 16)
                eb[r, sl] = jnp.maximum(
                    ewb[r, sl] + hb[r, pl.ds(cbase + g * 16, 16)], 0.0)

    def step(t, b):
        s, ns = slots[b], slots[1 - b]
        g_wait(s)
        a_wait(ns)
        g_issue(ns)

        @pl.when(t >= 2)
        def _():
            s_wait(s)

        compute(t, s)
        s_issue(s)

        @pl.when(t <= _TMAX - 3)
        def _():
            a_issue(t + 2, s)

    a_issue(0, slots[0])
    a_issue(1, slots[1])
    a_wait(slots[0])
    g_issue(slots[0])

    def pair_body(u, carry):
        t = u * 2
        step(t, 0)
        step(t + 1, 1)
        return carry

    lax.fori_loop(0, (_TMAX - 1) // 2, pair_body, 0)

    s0 = slots[0]
    g_wait(s0)
    s_wait(s0)
    compute(_TMAX - 1, s0)
    s_issue(s0)
    s_wait(slots[1])
    s_wait(slots[0])

    plsc.subcore_barrier()
    pltpu.sync_copy(seg.at[pl.ds(sid * _RPT, _RPT)],
                    out_hbm.at[cid, pl.ds(sid * _RPT, _RPT)])


def kernel(x, edge_index, edge_attr, u, batch,
           W1a, b1a, W1b, b1b, W2a, b2a, W2b, b2b):
    del u, batch
    h1 = _preh_call(x, W1a[:_DN])
    ew = _pree_call(edge_attr, W1a[_DN:], b1a.reshape(1, _H))
    part = _sc_edge(h1, ew, edge_index)
    return _epi_call(part, x, W1b, b1b.reshape(1, _H), W2a,
                     b2a.reshape(1, _H), W2b, b2b.reshape(1, _DN))


# speedup vs baseline: 3.4350x; 1.3099x over previous
import functools

import jax
import jax.numpy as jnp
from jax import lax
from jax.experimental import pallas as pl
from jax.experimental.pallas import tpu as pltpu
from jax.experimental.pallas import tpu_sc as plsc

_N = 10000
_E = 320000
_DN = 128
_DE = 16
_H = 128

_CH = 128
_PW = 72
_HF = 64
_NC = 2
_NS = 16
_NPAD = 10240
_RPT = _NPAD // _NS
_ZR = 128
_NCH = _E // _CH
_TMAX = 157
_DUMP = _NPAD - 1


def _preh_body(x_ref, w_ref, h_ref):
    h_ref[...] = jnp.dot(x_ref[...], w_ref[...],
                         preferred_element_type=jnp.float32)


def _preh_call(x, w):
    blk = 1000
    return pl.pallas_call(
        _preh_body,
        grid=(_N // blk,),
        in_specs=[pl.BlockSpec((blk, _DN), lambda i: (i, 0)),
                  pl.BlockSpec((_DN, _H), lambda i: (0, 0))],
        out_specs=pl.BlockSpec((blk, _H), lambda i: (i, 0)),
        out_shape=jax.ShapeDtypeStruct((_N, _H), jnp.float32),
    )(x, w)


def _pree_body(et_ref, w_ref, b_ref, o_ref):
    o_ref[...] = lax.dot_general(
        et_ref[...], w_ref[...], (((0,), (0,)), ((), ())),
        preferred_element_type=jnp.float32) + b_ref[...]


def _pree_call(et, w, b):
    blk = 16000
    return pl.pallas_call(
        _pree_body,
        grid=(_E // blk,),
        in_specs=[pl.BlockSpec((_DE, blk), lambda i: (0, i)),
                  pl.BlockSpec((_DE, _H), lambda i: (0, 0)),
                  pl.BlockSpec((1, _H), lambda i: (0, 0))],
        out_specs=pl.BlockSpec((blk, _H), lambda i: (i, 0)),
        out_shape=jax.ShapeDtypeStruct((_E, _H), jnp.float32),
    )(et, w, b)


def _epi_body(p_ref, x_ref, w1b_ref, b1b_ref, w2a_ref, b2a_ref, w2b_ref,
              b2b_ref, o_ref):
    sa = p_ref[0]
    sb = p_ref[1]
    cnt = sa[:, _HF:_HF + 1]
    ssum = jnp.concatenate([sa[:, :_HF], sb[:, :_HF]], axis=1)
    mean = ssum / jnp.maximum(cnt, 1.0)
    agg = (jnp.dot(mean, w1b_ref[...], preferred_element_type=jnp.float32)
           + b1b_ref[...] * (cnt > 0.0).astype(jnp.float32))
    xb = x_ref[...]
    h = jnp.maximum(
        jnp.dot(xb, w2a_ref[:_DN], preferred_element_type=jnp.float32)
        + jnp.dot(agg, w2a_ref[_DN:], preferred_element_type=jnp.float32)
        + b2a_ref[...], 0.0)
    o_ref[...] = (jnp.dot(h, w2b_ref[...], preferred_element_type=jnp.float32)
                  + b2b_ref[...] + xb)


def _epi_call(part, x, w1b, b1b, w2a, b2a, w2b, b2b):
    blk = 1000
    return pl.pallas_call(
        _epi_body,
        grid=(_N // blk,),
        in_specs=[pl.BlockSpec((2, blk, _PW), lambda i: (0, i, 0)),
                  pl.BlockSpec((blk, _DN), lambda i: (i, 0)),
                  pl.BlockSpec((_H, _H), lambda i: (0, 0)),
                  pl.BlockSpec((1, _H), lambda i: (0, 0)),
                  pl.BlockSpec((_H + _DN, _H), lambda i: (0, 0)),
                  pl.BlockSpec((1, _H), lambda i: (0, 0)),
                  pl.BlockSpec((_H, _DN), lambda i: (0, 0)),
                  pl.BlockSpec((1, _DN), lambda i: (0, 0))],
        out_specs=pl.BlockSpec((blk, _DN), lambda i: (i, 0)),
        out_shape=jax.ShapeDtypeStruct((_N, _DN), jnp.float32),
    )(part, x, w1b, b1b, w2a, b2a, w2b, b2b)


_MESH = plsc.VectorSubcoreMesh(core_axis_name="c", subcore_axis_name="s")


@functools.partial(
    pl.kernel,
    out_type=jax.ShapeDtypeStruct((_NC, _NPAD, _PW), jnp.float32),
    mesh=_MESH,
    scratch_types=[
        pltpu.VMEM((_CH,), jnp.int32),
        pltpu.VMEM((_CH,), jnp.int32),
        pltpu.VMEM((_CH,), jnp.int32),
        pltpu.VMEM((_CH,), jnp.int32),
        pltpu.VMEM((_CH,), jnp.int32),
        pltpu.VMEM((_CH,), jnp.int32),
        pltpu.VMEM((_CH, _HF), jnp.float32),
        pltpu.VMEM((_CH, _HF), jnp.float32),
        pltpu.VMEM((_CH, _H), jnp.float32),
        pltpu.VMEM((_CH, _H), jnp.float32),
        pltpu.VMEM((_CH, _PW), jnp.float32),
        pltpu.VMEM((_CH, _PW), jnp.float32),
        pltpu.VMEM_SHARED((_NPAD, _PW), jnp.float32),
        pltpu.SemaphoreType.DMA,
        pltpu.SemaphoreType.DMA,
        pltpu.SemaphoreType.DMA,
        pltpu.SemaphoreType.DMA,
        pltpu.SemaphoreType.DMA,
        pltpu.SemaphoreType.DMA,
    ],
    compiler_params=pltpu.CompilerParams(use_tc_tiling_on_sc=False))
def _sc_edge(h1_hbm, ew_hbm, ei_hbm, out_hbm,
             colv0, colv1, rowa0, rowa1, rows0, rows1, ewb0, ewb1,
             hb0, hb1, eb0, eb1, seg,
             semA0, semA1, semG0, semG1, semS0, semS1):
    cid = lax.axis_index("c")
    sid = lax.axis_index("s")
    cbase = cid * _HF

    slots = ((colv0, rowa0, rows0, ewb0, hb0, eb0, semA0, semG0, semS0),
             (colv1, rowa1, rows1, ewb1, hb1, eb1, semA1, semG1, semS1))

    def zrow(r, carry):
        for g in range(_PW // 16):
            eb0[r, pl.ds(g * 16, 16)] = jnp.zeros((16,), jnp.float32)
        return carry

    lax.fori_loop(0, _CH, zrow, 0)
    for k in range(_RPT // _ZR):
        pltpu.sync_copy(eb0, seg.at[pl.ds(sid * _RPT + k * _ZR, _ZR)])

    cpad = jnp.where(lax.iota(jnp.int32, 16) == 8, 1.0, 0.0)

    def crow(r, carry):
        eb0[r, pl.ds(_PW - 16, 16)] = cpad
        eb1[r, pl.ds(_PW - 16, 16)] = cpad
        return carry

    lax.fori_loop(0, _CH, crow, 0)
    plsc.subcore_barrier()

    def a_issue(t, s):
        colv, rowa, rows_, ewb, hb, eb, semA, semG, semS = s
        off = jnp.minimum((sid + t * _NS) * _CH, _E - _CH)
        pltpu.async_copy(ei_hbm.at[1, pl.ds(off, _CH)], colv, semA)
        pltpu.async_copy(ei_hbm.at[0, pl.ds(off, _CH)], rowa, semA)
        pltpu.async_copy(ew_hbm.at[pl.ds(off, _CH), pl.ds(cbase, _HF)],
                         ewb, semA)

    def a_wait(s):
        colv, rowa, rows_, ewb, hb, eb, semA, semG, semS = s
        pltpu.make_async_copy(ei_hbm.at[1, pl.ds(0, _CH)], colv, semA).wait()
        pltpu.make_async_copy(ei_hbm.at[0, pl.ds(0, _CH)], rowa, semA).wait()
        pltpu.make_async_copy(ew_hbm.at[pl.ds(0, _CH), pl.ds(cbase, _HF)],
                              ewb, semA).wait()

    def g_issue(s):
        colv, rowa, rows_, ewb, hb, eb, semA, semG, semS = s
        pltpu.async_copy(h1_hbm.at[colv], hb, semG)

    def g_wait(s):
        colv, rowa, rows_, ewb, hb, eb, semA, semG, semS = s
        pltpu.make_async_copy(h1_hbm.at[colv], hb, semG).wait()

    def s_issue(s):
        colv, rowa, rows_, ewb, hb, eb, semA, semG, semS = s
        pltpu.async_copy(eb, seg.at[rows_], semS, add=True)

    def s_wait(s):
        colv, rowa, rows_, ewb, hb, eb, semA, semG, semS = s
        pltpu.make_async_copy(eb, seg.at[rows_], semS).wait()

    def compute(t, s):
        colv, rowa, rows_, ewb, hb, eb, semA, semG, semS = s
        is_pad = (sid + t * _NS) >= _NCH
        dump = jnp.full((16,), _DUMP, jnp.int32)
        for g in range(_CH // 16):
            sl = pl.ds(g * 16, 16)
            rows_[sl] = jnp.where(is_pad, dump, rowa[sl])

        @plsc.parallel_loop(0, _CH, unroll=4)
        def rbody(r):
            for g in range(_HF // 16):
                sl = pl.ds(g * 16, 16)
                eb[r, sl] = jnp.maximum(
                    ewb[r, sl] + hb[r, pl.ds(cbase + g * 16, 16)], 0.0)

    def step(t, b):
        s, ns = slots[b], slots[1 - b]
        g_wait(s)
        a_wait(ns)
        g_issue(ns)

        @pl.when(t >= 2)
        def _():
            s_wait(s)

        compute(t, s)
        s_issue(s)

        @pl.when(t <= _TMAX - 3)
        def _():
            a_issue(t + 2, s)

    a_issue(0, slots[0])
    a_issue(1, slots[1])
    a_wait(slots[0])
    g_issue(slots[0])

    def pair_body(u, carry):
        t = u * 2
        step(t, 0)
        step(t + 1, 1)
        return carry

    lax.fori_loop(0, (_TMAX - 1) // 2, pair_body, 0)

    s0 = slots[0]
    g_wait(s0)
    s_wait(s0)
    compute(_TMAX - 1, s0)
    s_issue(s0)
    s_wait(slots[1])
    s_wait(slots[0])

    plsc.subcore_barrier()
    pltpu.sync_copy(seg.at[pl.ds(sid * _RPT, _RPT)],
                    out_hbm.at[cid, pl.ds(sid * _RPT, _RPT)])


def kernel(x, edge_index, edge_attr, u, batch,
           W1a, b1a, W1b, b1b, W2a, b2a, W2b, b2b):
    del u, batch
    h1 = _preh_call(x, W1a[:_DN])
    ew = _pree_call(edge_attr.T, W1a[_DN:], b1a.reshape(1, _H))
    part = _sc_edge(h1, ew, edge_index)
    return _epi_call(part, x, W1b, b1b.reshape(1, _H), W2a,
                     b2a.reshape(1, _H), W2b, b2b.reshape(1, _DN))
